# baseline (speedup 1.0000x reference)
import functools

import jax
import jax.numpy as jnp
from jax import lax
from jax.experimental import pallas as pl
from jax.experimental.pallas import tpu as pltpu
from jax.experimental.pallas import tpu_sc as plsc

_B = 4
_S = 4096
_D = 128
_EPS = 1e-12

_N = _B * _S
_NW = 32
_CH = 128

_NCHUNK = 2
_NTOK = _N // _NCHUNK

_ROWS_PER_BLK = 2048


def _sc_gather(idx2d, table, n_tokens):
    tok_per_w = n_tokens // _NW
    nch = tok_per_w // _CH
    mesh = plsc.VectorSubcoreMesh(core_axis_name="c", subcore_axis_name="s")

    @functools.partial(
        pl.kernel,
        mesh=mesh,
        out_type=jax.ShapeDtypeStruct((n_tokens, _D), jnp.float32),
        scratch_types=[
            pltpu.VMEM((nch, _CH), jnp.int32),
            pltpu.VMEM((tok_per_w, _D), jnp.float32),
            pltpu.SemaphoreType.DMA,
        ],
    )
    def gather_kernel(idx_hbm, table_hbm, out_hbm, idx_v, rows_v, sem):
        wid = lax.axis_index("s") * 2 + lax.axis_index("c")
        pltpu.sync_copy(idx_hbm.at[pl.ds(wid * nch, nch)], idx_v)
        copies = [
            pltpu.async_copy(
                table_hbm.at[idx_v.at[j]],
                rows_v.at[pl.ds(j * _CH, _CH)],
                sem,
            )
            for j in range(nch)
        ]
        for c in copies:
            c.wait()
        pltpu.sync_copy(rows_v, out_hbm.at[pl.ds(wid * tok_per_w, tok_per_w)])

    return gather_kernel(idx2d, table)


def _tc_ln_body(x_ref, tt_ref, tte_ref, g_ref, b_ref, o_ref):
    x = x_ref[...]
    tt = tt_ref[...]
    tte = tte_ref[...]
    x = x + jnp.where(tt > 0, tte[1:2, :], tte[0:1, :])
    mean = jnp.mean(x, axis=-1, keepdims=True)
    xm = x - mean
    var = jnp.mean(xm * xm, axis=-1, keepdims=True)
    inv = lax.rsqrt(var + _EPS)
    o_ref[...] = xm * inv * g_ref[...] + b_ref[...]


def _tc_layernorm(x, tt, tte, gamma, beta, n_tokens, interpret=False):
    grid = (n_tokens // _ROWS_PER_BLK,)
    return pl.pallas_call(
        _tc_ln_body,
        grid=grid,
        in_specs=[
            pl.BlockSpec((_ROWS_PER_BLK, _D), lambda i: (i, 0)),
            pl.BlockSpec((_ROWS_PER_BLK, 1), lambda i: (i, 0)),
            pl.BlockSpec((2, _D), lambda i: (0, 0)),
            pl.BlockSpec((1, _D), lambda i: (0, 0)),
            pl.BlockSpec((1, _D), lambda i: (0, 0)),
        ],
        out_specs=pl.BlockSpec((_ROWS_PER_BLK, _D), lambda i: (i, 0)),
        out_shape=jax.ShapeDtypeStruct((n_tokens, _D), jnp.float32),
        interpret=interpret,
    )(x, tt, tte, gamma, beta)


def kernel(input_ids, token_type_ids, word_embeddings, token_type_embeddings,
           ln_gamma, ln_beta):
    ids = input_ids.reshape(_N)
    tt_all = token_type_ids.reshape(_N, 1)
    gamma = ln_gamma.reshape(1, _D)
    beta = ln_beta.reshape(1, _D)
    outs = []
    for c in range(_NCHUNK):
        idx2d = lax.slice_in_dim(ids, c * _NTOK, (c + 1) * _NTOK).reshape(
            _NTOK // _CH, _CH)
        gathered = _sc_gather(idx2d, word_embeddings, _NTOK)
        tt = lax.slice_in_dim(tt_all, c * _NTOK, (c + 1) * _NTOK)
        outs.append(_tc_layernorm(
            gathered, tt, token_type_embeddings, gamma, beta, _NTOK))
    return jnp.concatenate(outs, axis=0).reshape(_B, _S, _D)


# speedup vs baseline: 1.3361x; 1.3361x over previous
import functools

import jax
import jax.numpy as jnp
from jax import lax
from jax.experimental import pallas as pl
from jax.experimental.pallas import tpu as pltpu
from jax.experimental.pallas import tpu_sc as plsc

_B = 4
_S = 4096
_D = 128
_EPS = 1e-12

_N = _B * _S
_NW = 32
_CH = 128

_NCHUNK = 2
_NTOK = _N // _NCHUNK

_ROWS_PER_BLK = 2048


def _sc_gather(idx2d, table, n_tokens):
    tok_per_w = n_tokens // _NW
    nch = tok_per_w // _CH
    mesh = plsc.VectorSubcoreMesh(core_axis_name="c", subcore_axis_name="s")

    @functools.partial(
        pl.kernel,
        mesh=mesh,
        out_type=jax.ShapeDtypeStruct((n_tokens, _D), jnp.float32),
        scratch_types=[
            pltpu.VMEM((nch, _CH), jnp.int32),
            pltpu.VMEM((tok_per_w, _D), jnp.float32),
            pltpu.SemaphoreType.DMA,
        ],
    )
    def gather_kernel(idx_hbm, table_hbm, out_hbm, idx_v, rows_v, sem):
        wid = lax.axis_index("s") * 2 + lax.axis_index("c")
        pltpu.sync_copy(idx_hbm.at[pl.ds(wid * nch, nch)], idx_v)
        copies = [
            pltpu.async_copy(
                table_hbm.at[idx_v.at[j]],
                rows_v.at[pl.ds(j * _CH, _CH)],
                sem,
            )
            for j in range(nch)
        ]
        for c in copies:
            c.wait()
        pltpu.sync_copy(rows_v, out_hbm.at[pl.ds(wid * tok_per_w, tok_per_w)])

    return gather_kernel(idx2d, table)


def _tc_ln_body(x_ref, tt_ref, tte_ref, g_ref, b_ref, o_ref):
    x = x_ref[...]
    blk = x.shape[0]
    ttf = tt_ref[...].astype(jnp.float32)
    tte = tte_ref[...]
    row0 = lax.broadcast_in_dim(tte[0], (1, 1, _D), (2,))
    row1 = lax.broadcast_in_dim(tte[1], (1, 1, _D), (2,))
    big = lax.broadcast_in_dim(ttf, (blk, 128, 128), (0, 2))
    ic = lax.broadcasted_iota(jnp.int32, (1, 128, 128), 1)
    ik = lax.broadcasted_iota(jnp.int32, (1, 128, 128), 2)
    eyef = (ic == ik).astype(jnp.float32)
    tt3 = jnp.sum(big * eyef, axis=-1, keepdims=True)
    x = x + row0 + tt3 * (row1 - row0)
    mean = jnp.mean(x, axis=-1, keepdims=True)
    xm = x - mean
    var = jnp.mean(xm * xm, axis=-1, keepdims=True)
    inv = lax.rsqrt(var + _EPS)
    g = lax.broadcast_in_dim(g_ref[...][0], (1, 1, _D), (2,))
    b = lax.broadcast_in_dim(b_ref[...][0], (1, 1, _D), (2,))
    o_ref[...] = xm * inv * g + b


def _tc_layernorm(x3, tt2, tte, gamma, beta, n_tokens, interpret=False):
    rows = n_tokens // 128
    blk = 8
    grid = (rows // blk,)
    return pl.pallas_call(
        _tc_ln_body,
        grid=grid,
        in_specs=[
            pl.BlockSpec((blk, 128, _D), lambda i: (i, 0, 0)),
            pl.BlockSpec((blk, 128), lambda i: (i, 0)),
            pl.BlockSpec((2, _D), lambda i: (0, 0)),
            pl.BlockSpec((1, _D), lambda i: (0, 0)),
            pl.BlockSpec((1, _D), lambda i: (0, 0)),
        ],
        out_specs=pl.BlockSpec((blk, 128, _D), lambda i: (i, 0, 0)),
        out_shape=jax.ShapeDtypeStruct((rows, 128, _D), jnp.float32),
        interpret=interpret,
    )(x3, tt2, tte, gamma, beta)


def kernel(input_ids, token_type_ids, word_embeddings, token_type_embeddings,
           ln_gamma, ln_beta):
    idx2d = input_ids.reshape(_N // _CH, _CH)
    gathered = _sc_gather(idx2d, word_embeddings, _N)
    x3 = gathered.reshape(_N // 128, 128, _D)
    tt2 = token_type_ids.reshape(_N // 128, 128)
    out = _tc_layernorm(
        x3, tt2, token_type_embeddings,
        ln_gamma.reshape(1, _D), ln_beta.reshape(1, _D), _N,
    )
    return out.reshape(_B, _S, _D)
